# baseline (speedup 1.0000x reference)
import functools

import jax
import jax.numpy as jnp
from jax import lax
from jax.experimental import pallas as pl
from jax.experimental.pallas import tpu as pltpu
from jax.experimental.pallas import tpu_sc as plsc

_N = 256
_R = 4
_L = 2


def _make_emb_kernel(n_t_total, n_t, emb_dim):
    mesh = plsc.VectorSubcoreMesh(core_axis_name="c", subcore_axis_name="s")
    num_cores = mesh.num_cores

    @functools.partial(
        pl.kernel,
        out_type=jax.ShapeDtypeStruct((n_t_total * _N, emb_dim), jnp.float32),
        mesh=mesh,
        scratch_types=[
            pltpu.VMEM((n_t, _N), jnp.int32),
            [pltpu.VMEM((_N, emb_dim), jnp.float32) for _ in range(_R)],
            [pltpu.SemaphoreType.DMA for _ in range(_R)],
            [pltpu.SemaphoreType.DMA for _ in range(_R)],
        ],
        compiler_params=pltpu.CompilerParams(use_tc_tiling_on_sc=False),
    )
    def emb(x_hbm, table_hbm, out_hbm, idx_v, rows, gs, ps):
        wid = lax.axis_index("s") * num_cores + lax.axis_index("c")
        t0 = wid * n_t
        pltpu.sync_copy(x_hbm.at[pl.ds(t0, n_t)], idx_v)

        def gather(t, b):
            return pltpu.async_copy(
                table_hbm.at[idx_v.at[t]], rows[b], gs[b]
            )

        def wait_gather(t, b):
            pltpu.make_async_copy(
                table_hbm.at[idx_v.at[t]], rows[b], gs[b]
            ).wait()

        def put(t, b):
            return pltpu.async_copy(
                rows[b], out_hbm.at[pl.ds((t0 + t) * _N, _N)], ps[b]
            )

        def wait_put(b):
            pltpu.make_async_copy(
                rows[b], out_hbm.at[pl.ds(t0 * _N, _N)], ps[b]
            ).wait()

        for b in range(_L):
            gather(b, b)

        def group(k, carry):
            for b in range(_R):
                t = k * _R + b
                wait_gather(t, b)
                put(t, b)
                nb = (b + _L) % _R
                if b < _R - _L:
                    @pl.when(k > 0)
                    def _():
                        wait_put(nb)
                else:
                    wait_put(nb)
                gather(t + _L, nb)
            return carry

        n_groups = n_t // _R
        lax.fori_loop(0, n_groups - 1, group, 0, unroll=False)

        for b in range(_R):
            t = (n_groups - 1) * _R + b
            wait_gather(t, b)
            put(t, b)
            if b < _R - _L:
                wait_put((b + _L) % _R)
                gather(t + _L, (b + _L) % _R)

        for b in range(_R):
            wait_put(b)

    return emb


def kernel(x, table):
    b0, b1 = x.shape
    vocab, emb_dim = table.shape
    n = b0 * b1
    n_t_total = n // _N
    n_workers = 32
    n_t = n_t_total // n_workers
    xf = x.reshape(n_t_total, _N).astype(jnp.int32)
    emb = _make_emb_kernel(n_t_total, n_t, emb_dim)
    out = emb(xf, table)
    return out.reshape(b0, b1, emb_dim)
